# baseline (speedup 1.0000x reference)
import functools

import jax
import jax.numpy as jnp
from jax import lax
from jax.experimental import pallas as pl
from jax.experimental.pallas import tpu as pltpu
from jax.experimental.pallas import tpu_sc as plsc

D = 384
K = 8192
BK = 1024
CCW = (0.05, 0.25, 0.6)

NC = 2
NS = 16
NW = NC * NS

_DOT_DTYPE = jnp.bfloat16


RG = 32
C = BK // 128


def _argmin_body(zb2_ref, znorm_ref, cb_ref, cbnorm_ref, idx_ref, part_ref,
                 mm, accv, acci):
    k = pl.program_id(1)
    nk = pl.num_programs(1)
    bn = zb2_ref.shape[0]
    cbb = cb_ref[...].astype(_DOT_DTYPE)
    mm[...] = lax.dot_general(zb2_ref[...], cbb, (((1,), (1,)), ((), ())),
                              preferred_element_type=jnp.float32)
    cbn = cbnorm_ref[...]
    cbn_chunks = [cbn[:, c * 128:(c + 1) * 128] for c in range(C)]

    @pl.when(k == 0)
    def _():
        accv[...] = jnp.full((bn, 128), jnp.inf, jnp.float32)
        acci[...] = jnp.zeros((bn, 128), jnp.int32)

    def grp(g, _):
        r0 = g * RG
        zn = znorm_ref[pl.ds(r0, RG), :]
        av = accv[pl.ds(r0, RG), :]
        ai = acci[pl.ds(r0, RG), :]
        for c in range(C):
            d = mm[pl.ds(r0, RG), pl.ds(c * 128, 128)]
            dist = (zn + d) + cbn_chunks[c]
            msk = dist < av
            av = jnp.where(msk, dist, av)
            ai = jnp.where(msk, k * C + c, ai)
        accv[pl.ds(r0, RG), :] = av
        acci[pl.ds(r0, RG), :] = ai
        return 0

    lax.fori_loop(0, bn // RG, grp, 0)

    @pl.when(k == nk - 1)
    def _():
        def fin(g, s):
            r0 = g * RG
            av = accv[pl.ds(r0, RG), :]
            ai = acci[pl.ds(r0, RG), :]
            m = jnp.min(av, axis=1, keepdims=True)
            lane = lax.broadcasted_iota(jnp.int32, (RG, 128), 1)
            kg = ai * 128 + lane
            loc = jnp.min(jnp.where(av == m, kg, jnp.int32(2 ** 30)),
                          axis=1, keepdims=True)
            idx_ref[pl.ds(r0, RG), :] = loc
            return s + jnp.sum(m)

        s = lax.fori_loop(0, bn // RG, fin, jnp.float32(0.0))
        part_ref[...] = jnp.full((1, 1, 1), s, jnp.float32)


def _argmin_call(zb2, znorm, cb, cbnorm, bn, interpret=False):
    n = zb2.shape[0]
    nrb = n // bn
    nk = K // BK
    return pl.pallas_call(
        _argmin_body,
        grid=(nrb, nk),
        in_specs=[
            pl.BlockSpec((bn, D), lambda r, k: (r, 0)),
            pl.BlockSpec((bn, 1), lambda r, k: (r, 0)),
            pl.BlockSpec((BK, D), lambda r, k: (k, 0)),
            pl.BlockSpec((1, BK), lambda r, k: (0, k)),
        ],
        out_specs=[
            pl.BlockSpec((bn, 1), lambda r, k: (r, 0)),
            pl.BlockSpec((1, 1, 1), lambda r, k: (r, 0, 0)),
        ],
        out_shape=[
            jax.ShapeDtypeStruct((n, 1), jnp.int32),
            jax.ShapeDtypeStruct((nrb, 1, 1), jnp.float32),
        ],
        scratch_shapes=[
            pltpu.VMEM((bn, BK), jnp.float32),
            pltpu.VMEM((bn, 128), jnp.float32),
            pltpu.VMEM((bn, 128), jnp.int32),
        ],
        interpret=interpret,
    )(zb2, znorm, cb, cbnorm)


@functools.lru_cache(maxsize=None)
def _make_gather(n):
    b_per_w = n // NW
    mesh = plsc.VectorSubcoreMesh(core_axis_name="c", subcore_axis_name="s")

    @functools.partial(
        pl.kernel,
        mesh=mesh,
        out_type=jax.ShapeDtypeStruct((n, D), jnp.float32),
        scratch_types=[
            pltpu.VMEM((b_per_w,), jnp.int32),
            pltpu.VMEM((b_per_w, D), jnp.float32),
            pltpu.SemaphoreType.DMA,
        ],
    )
    def gather(cb_hbm, idx_hbm, out_hbm, idx_v, rows_v, sem):
        wid = lax.axis_index("s") * NC + lax.axis_index("c")
        base = wid * b_per_w
        pltpu.sync_copy(idx_hbm.at[pl.ds(base, b_per_w)], idx_v)
        pltpu.async_copy(cb_hbm.at[idx_v], rows_v, sem).wait()
        pltpu.sync_copy(rows_v, out_hbm.at[pl.ds(base, b_per_w)])

    return gather


def kernel(l0, l1, l2, cb0, cb1, cb2):
    out = []
    for i, (z, cb, bn) in enumerate(((l0, cb0, 1024), (l1, cb1, 2048),
                                     (l2, cb2, 2048))):
        flat = z.reshape(-1, D)
        n = flat.shape[0]
        znorm = jnp.sum(flat ** 2, axis=1, keepdims=True)
        cbnorm = jnp.sum(cb ** 2, axis=1)[None, :]
        zb2 = (-2.0 * flat).astype(_DOT_DTYPE)
        idx2d, part = _argmin_call(zb2, znorm, cb, cbnorm, bn)
        idx = idx2d.reshape(z.shape[:-1])
        q = _make_gather(n)(cb, idx2d.reshape(-1)).reshape(z.shape)
        loss = jnp.float32(CCW[i]) * (jnp.sum(part) / jnp.float32(n * D))
        out.append((idx, loss, q))
    (idx0, loss0, q0), (idx1, loss1, q1), (idx2_, loss2, q2) = out
    total = loss0 + loss1 + loss2
    return (idx0, idx1, idx2_, total, q0, q1, q2)


# speedup vs baseline: 1.5337x; 1.5337x over previous
import functools

import jax
import jax.numpy as jnp
from jax import lax
from jax.experimental import pallas as pl
from jax.experimental.pallas import tpu as pltpu
from jax.experimental.pallas import tpu_sc as plsc

D = 384
K = 8192
BK = 1024
CCW = (0.05, 0.25, 0.6)

NC = 2
NS = 16
NW = NC * NS

_DOT_DTYPE = jnp.bfloat16


def _argmin_body(ids_ref, zb2_ref, znorm_ref, cb_ref, cbnorm_ref,
                 idx_ref, part_ref, mma, mmb, accv, acci):
    k = pl.program_id(1)
    nk = pl.num_programs(1) - 1

    @pl.when(k < nk)
    def _():
        cbb = cb_ref[...].astype(_DOT_DTYPE)
        prod = lax.dot_general(zb2_ref[...], cbb, (((1,), (1,)), ((), ())),
                               preferred_element_type=jnp.float32)

        @pl.when(k % 2 == 0)
        def _():
            mma[...] = prod

        @pl.when(k % 2 == 1)
        def _():
            mmb[...] = prod

    @pl.when(k > 0)
    def _():
        kk = k - 1

        def consume(buf_ref):
            dist = (znorm_ref[...] + buf_ref[...]) + cbnorm_ref[...]
            m = jnp.min(dist, axis=1, keepdims=True)
            loc = jnp.min(jnp.where(dist == m, ids_ref[...], K),
                          axis=1, keepdims=True) + kk * BK

            @pl.when(kk == 0)
            def _():
                accv[...] = m
                acci[...] = loc

            @pl.when(kk > 0)
            def _():
                better = m < accv[...]
                accv[...] = jnp.where(better, m, accv[...])
                acci[...] = jnp.where(better, loc, acci[...])

        @pl.when(kk % 2 == 0)
        def _():
            consume(mma)

        @pl.when(kk % 2 == 1)
        def _():
            consume(mmb)

    @pl.when(k == nk)
    def _():
        idx_ref[...] = acci[...]
        part_ref[...] = jnp.sum(accv[...], keepdims=True)[None]


def _argmin_call(ids, zb2, znorm, cb, cbnorm, interpret=False):
    n = zb2.shape[0]
    bn = min(n, 2048)
    nrb = n // bn
    nk = K // BK
    return pl.pallas_call(
        _argmin_body,
        grid=(nrb, nk + 1),
        in_specs=[
            pl.BlockSpec((1, BK), lambda r, k: (0, 0)),
            pl.BlockSpec((bn, D), lambda r, k: (r, 0)),
            pl.BlockSpec((bn, 1), lambda r, k: (r, 0)),
            pl.BlockSpec((BK, D), lambda r, k: (jnp.minimum(k, nk - 1), 0)),
            pl.BlockSpec((1, BK), lambda r, k: (0, jnp.maximum(k - 1, 0))),
        ],
        out_specs=[
            pl.BlockSpec((bn, 1), lambda r, k: (r, 0)),
            pl.BlockSpec((1, 1, 1), lambda r, k: (r, 0, 0)),
        ],
        out_shape=[
            jax.ShapeDtypeStruct((n, 1), jnp.int32),
            jax.ShapeDtypeStruct((nrb, 1, 1), jnp.float32),
        ],
        scratch_shapes=[
            pltpu.VMEM((bn, BK), jnp.float32),
            pltpu.VMEM((bn, BK), jnp.float32),
            pltpu.VMEM((bn, 1), jnp.float32),
            pltpu.VMEM((bn, 1), jnp.int32),
        ],
        interpret=interpret,
    )(ids, zb2, znorm, cb, cbnorm)


@functools.lru_cache(maxsize=None)
def _make_gather(n):
    b_per_w = n // NW
    mesh = plsc.VectorSubcoreMesh(core_axis_name="c", subcore_axis_name="s")

    @functools.partial(
        pl.kernel,
        mesh=mesh,
        out_type=jax.ShapeDtypeStruct((n, D), jnp.float32),
        scratch_types=[
            pltpu.VMEM((b_per_w,), jnp.int32),
            pltpu.VMEM((b_per_w, D), jnp.float32),
            pltpu.SemaphoreType.DMA,
        ],
    )
    def gather(cb_hbm, idx_hbm, out_hbm, idx_v, rows_v, sem):
        wid = lax.axis_index("s") * NC + lax.axis_index("c")
        base = wid * b_per_w
        pltpu.sync_copy(idx_hbm.at[pl.ds(base, b_per_w)], idx_v)
        pltpu.async_copy(cb_hbm.at[idx_v], rows_v, sem).wait()
        pltpu.sync_copy(rows_v, out_hbm.at[pl.ds(base, b_per_w)])

    return gather


def kernel(l0, l1, l2, cb0, cb1, cb2):
    ids = jnp.arange(BK, dtype=jnp.int32)[None, :]
    out = []
    for i, (z, cb) in enumerate(((l0, cb0), (l1, cb1), (l2, cb2))):
        flat = z.reshape(-1, D)
        n = flat.shape[0]
        znorm = jnp.sum(flat ** 2, axis=1, keepdims=True)
        cbnorm = jnp.sum(cb ** 2, axis=1)[None, :]
        zb2 = (-2.0 * flat).astype(_DOT_DTYPE)
        idx2d, part = _argmin_call(ids, zb2, znorm, cb, cbnorm)
        idx = idx2d.reshape(z.shape[:-1])
        q = _make_gather(n)(cb, idx2d.reshape(-1)).reshape(z.shape)
        loss = jnp.float32(CCW[i]) * (jnp.sum(part) / jnp.float32(n * D))
        out.append((idx, loss, q))
    (idx0, loss0, q0), (idx1, loss1, q1), (idx2_, loss2, q2) = out
    total = loss0 + loss1 + loss2
    return (idx0, idx1, idx2_, total, q0, q1, q2)


# speedup vs baseline: 2.0521x; 1.3380x over previous
import functools

import jax
import jax.numpy as jnp
from jax import lax
from jax.experimental import pallas as pl
from jax.experimental.pallas import tpu as pltpu
from jax.experimental.pallas import tpu_sc as plsc

D = 384
K = 8192
BK = 8192
CCW = (0.05, 0.25, 0.6)

NC = 2
NS = 16
NW = NC * NS

_DOT_DTYPE = jnp.bfloat16


def _argmin_body(ids_ref, zb2_ref, znorm_ref, cb_ref, cbnorm_ref,
                 idx_ref, part_ref, accv, acci):
    k = pl.program_id(1)
    nk = pl.num_programs(1)
    cbb = cb_ref[...].astype(_DOT_DTYPE)
    m2 = lax.dot_general(zb2_ref[...], cbb, (((1,), (1,)), ((), ())),
                         preferred_element_type=jnp.float32)
    dist = (znorm_ref[...] + m2) + cbnorm_ref[...]
    m = jnp.min(dist, axis=1, keepdims=True)
    loc = jnp.min(jnp.where(dist == m, ids_ref[...], K),
                  axis=1, keepdims=True) + k * BK

    @pl.when(k == 0)
    def _():
        accv[...] = m
        acci[...] = loc

    @pl.when(k > 0)
    def _():
        better = m < accv[...]
        accv[...] = jnp.where(better, m, accv[...])
        acci[...] = jnp.where(better, loc, acci[...])

    @pl.when(k == nk - 1)
    def _():
        idx_ref[...] = acci[...]
        part_ref[...] = jnp.sum(accv[...], keepdims=True)[None]


def _argmin_call(ids, zb2, znorm, cb, cbnorm, interpret=False):
    n = zb2.shape[0]
    bn = min(n, 1024)
    nrb = n // bn
    nk = K // BK
    return pl.pallas_call(
        _argmin_body,
        grid=(nrb, nk),
        in_specs=[
            pl.BlockSpec((1, BK), lambda r, k: (0, 0)),
            pl.BlockSpec((bn, D), lambda r, k: (r, 0)),
            pl.BlockSpec((bn, 1), lambda r, k: (r, 0)),
            pl.BlockSpec((BK, D), lambda r, k: (k, 0)),
            pl.BlockSpec((1, BK), lambda r, k: (0, k)),
        ],
        out_specs=[
            pl.BlockSpec((bn, 1), lambda r, k: (r, 0)),
            pl.BlockSpec((1, 1, 1), lambda r, k: (r, 0, 0)),
        ],
        out_shape=[
            jax.ShapeDtypeStruct((n, 1), jnp.int32),
            jax.ShapeDtypeStruct((nrb, 1, 1), jnp.float32),
        ],
        scratch_shapes=[
            pltpu.VMEM((bn, 1), jnp.float32),
            pltpu.VMEM((bn, 1), jnp.int32),
        ],
        interpret=interpret,
    )(ids, zb2, znorm, cb, cbnorm)


@functools.lru_cache(maxsize=None)
def _make_gather(n):
    b_per_w = n // NW
    mesh = plsc.VectorSubcoreMesh(core_axis_name="c", subcore_axis_name="s")

    @functools.partial(
        pl.kernel,
        mesh=mesh,
        out_type=jax.ShapeDtypeStruct((n, D), jnp.float32),
        scratch_types=[
            pltpu.VMEM((b_per_w,), jnp.int32),
            pltpu.VMEM((b_per_w, D), jnp.float32),
            pltpu.SemaphoreType.DMA,
        ],
    )
    def gather(cb_hbm, idx_hbm, out_hbm, idx_v, rows_v, sem):
        wid = lax.axis_index("s") * NC + lax.axis_index("c")
        base = wid * b_per_w
        pltpu.sync_copy(idx_hbm.at[pl.ds(base, b_per_w)], idx_v)
        pltpu.async_copy(cb_hbm.at[idx_v], rows_v, sem).wait()
        pltpu.sync_copy(rows_v, out_hbm.at[pl.ds(base, b_per_w)])

    return gather


def kernel(l0, l1, l2, cb0, cb1, cb2):
    ids = jnp.arange(BK, dtype=jnp.int32)[None, :]
    out = []
    for i, (z, cb) in enumerate(((l0, cb0), (l1, cb1), (l2, cb2))):
        flat = z.reshape(-1, D)
        n = flat.shape[0]
        znorm = jnp.sum(flat ** 2, axis=1, keepdims=True)
        cbnorm = jnp.sum(cb ** 2, axis=1)[None, :]
        zb2 = (-2.0 * flat).astype(_DOT_DTYPE)
        idx2d, part = _argmin_call(ids, zb2, znorm, cb, cbnorm)
        idx = idx2d.reshape(z.shape[:-1])
        q = _make_gather(n)(cb, idx2d.reshape(-1)).reshape(z.shape)
        loss = jnp.float32(CCW[i]) * (jnp.sum(part) / jnp.float32(n * D))
        out.append((idx, loss, q))
    (idx0, loss0, q0), (idx1, loss1, q1), (idx2_, loss2, q2) = out
    total = loss0 + loss1 + loss2
    return (idx0, idx1, idx2_, total, q0, q1, q2)
